# Initial kernel scaffold; baseline (speedup 1.0000x reference)
import jax
import jax.numpy as jnp
from jax.experimental import pallas as pl


def kernel(x, edge_index, edge_attr, batch, params):
    raise NotImplementedError("write your pallas kernel here")


# speedup vs baseline: 1.6724x; 1.6724x over previous
import functools

import jax
import jax.numpy as jnp
from jax import lax
from jax.experimental import pallas as pl
from jax.experimental.pallas import tpu as pltpu
from jax.experimental.pallas import tpu_sc as plsc

N = 50000
G = 256
H = 64
DIN = 128
E = 800000

NB = 2048
NPAD = 51200
EB = 2048
EPAD = 819200
EROWS = EPAD // 128
NC, NS, LANES = 2, 16, 16
NWORK = NC * NS
EW = EPAD // NWORK
CHUNK = 512
NHALF = 25000
TRASH = 25088
ACCROWS = 25600

_f32 = jnp.float32
_i32 = jnp.int32


def _sc_mesh():
    return plsc.VectorSubcoreMesh(core_axis_name="c", subcore_axis_name="s")


def _edge_feat_stage(x0, x3, bf, row_p, col_p):
    @functools.partial(
        pl.kernel,
        out_type=(jax.ShapeDtypeStruct((EPAD,), _f32),
                  jax.ShapeDtypeStruct((EPAD,), _f32),
                  jax.ShapeDtypeStruct((EPAD,), _i32)),
        mesh=_sc_mesh(),
        compiler_params=pltpu.CompilerParams(needs_layout_passes=False, use_tc_tiling_on_sc=False),
        scratch_types=[
            pltpu.VMEM((NPAD,), _f32),
            pltpu.VMEM((NPAD,), _f32),
            pltpu.VMEM((CHUNK,), _i32),
            pltpu.VMEM((CHUNK,), _i32),
            pltpu.VMEM((CHUNK,), _f32),
            pltpu.VMEM((CHUNK,), _f32),
            pltpu.VMEM((CHUNK,), _i32),
        ],
    )
    def k(x0_h, x3_h, bf_h, row_h, col_h, d0_h, d1_h, be_h,
          t0, t1, ir, ic, d0v, d1v, bev):
        wid = lax.axis_index("s") * NC + lax.axis_index("c")
        pltpu.sync_copy(x0_h, t0)
        pltpu.sync_copy(x3_h, t1)

        def chunk(c, carry):
            ebase = pl.multiple_of(wid * EW + c * CHUNK, 8)
            pltpu.sync_copy(row_h.at[pl.ds(ebase, CHUNK)], ir)
            pltpu.sync_copy(col_h.at[pl.ds(ebase, CHUNK)], ic)
            for s in range(CHUNK // 16):
                ri = ir[pl.ds(s * 16, 16)]
                ci = ic[pl.ds(s * 16, 16)]
                d0v[pl.ds(s * 16, 16)] = (plsc.load_gather(t0, [ri])
                                          - plsc.load_gather(t0, [ci]))
                d1v[pl.ds(s * 16, 16)] = (plsc.load_gather(t1, [ri])
                                          - plsc.load_gather(t1, [ci]))
            pltpu.sync_copy(d0v, d0_h.at[pl.ds(ebase, CHUNK)])
            pltpu.sync_copy(d1v, d1_h.at[pl.ds(ebase, CHUNK)])
            return carry

        lax.fori_loop(0, EW // CHUNK, chunk, 0)

        pltpu.sync_copy(bf_h, t0)

        def chunk2(c, carry):
            ebase = pl.multiple_of(wid * EW + c * CHUNK, 8)
            pltpu.sync_copy(row_h.at[pl.ds(ebase, CHUNK)], ir)
            for s in range(CHUNK // 16):
                ri = ir[pl.ds(s * 16, 16)]
                bev[pl.ds(s * 16, 16)] = plsc.load_gather(
                    t0, [ri]).astype(_i32)
            pltpu.sync_copy(bev, be_h.at[pl.ds(ebase, CHUNK)])
            return carry

        lax.fori_loop(0, EW // CHUNK, chunk2, 0)

    return k(x0, x3, bf, row_p, col_p)


def _gather_stage(h, row2d, col2d):
    @functools.partial(
        pl.kernel,
        out_type=(jax.ShapeDtypeStruct((EPAD, H), _f32),
                  jax.ShapeDtypeStruct((EPAD, H), _f32)),
        mesh=_sc_mesh(),
        compiler_params=pltpu.CompilerParams(needs_layout_passes=False, use_tc_tiling_on_sc=False),
        scratch_types=[
            pltpu.VMEM((4, 128), _i32),
            pltpu.VMEM((4, 128), _i32),
            pltpu.VMEM((CHUNK, H), _f32),
            pltpu.VMEM((CHUNK, H), _f32),
            pltpu.SemaphoreType.DMA,
        ],
    )
    def k(h_h, row_h, col_h, hr_h, hc_h, idxa, idxb, bufa, bufb, sem):
        wid = lax.axis_index("s") * NC + lax.axis_index("c")

        def chunk(c, carry):
            crow = wid * (EW // 128) + c * 4
            ebase = pl.multiple_of(crow * 128, 8)
            pltpu.sync_copy(row_h.at[pl.ds(crow, 4)], idxa)
            pltpu.sync_copy(col_h.at[pl.ds(crow, 4)], idxb)
            for j in range(4):
                pltpu.async_copy(h_h.at[idxa.at[j]],
                                 bufa.at[pl.ds(j * 128, 128)], sem).wait()
                pltpu.async_copy(h_h.at[idxb.at[j]],
                                 bufb.at[pl.ds(j * 128, 128)], sem).wait()
            pltpu.sync_copy(bufa, hr_h.at[pl.ds(ebase, CHUNK)])
            pltpu.sync_copy(bufb, hc_h.at[pl.ds(ebase, CHUNK)])
            return carry

        lax.fori_loop(0, EW // CHUNK, chunk, 0)

    return k(h, row2d, col2d)


def _scatter_stage(m, col2d):
    @functools.partial(
        pl.kernel,
        out_type=jax.ShapeDtypeStruct((2, ACCROWS, H), _f32),
        mesh=_sc_mesh(),
        compiler_params=pltpu.CompilerParams(needs_layout_passes=False, use_tc_tiling_on_sc=False),
        scratch_types=[
            pltpu.VMEM_SHARED((ACCROWS, H), _f32),
            pltpu.VMEM((2, 128), _i32),
            pltpu.VMEM((256, H), _f32),
            pltpu.VMEM((160, H), _f32),
            pltpu.SemaphoreType.DMA,
        ],
    )
    def k(m_h, col_h, out_h, acc, idxv, mbuf, wbuf, sem):
        cid = lax.axis_index("c")
        sid = lax.axis_index("s")
        lo = cid * NHALF

        def zrow(i, carry):
            for q in range(H // 16):
                wbuf[i, pl.ds(q * 16, 16)] = jnp.zeros((16,), _f32)
            return carry
        lax.fori_loop(0, 160, zrow, 0)
        for q in range(10):
            pltpu.sync_copy(wbuf, acc.at[pl.ds(sid * 1600 + q * 160, 160)])
        plsc.subcore_barrier()

        nrows = EPAD // 128 // NS

        def chunk(c, carry):
            crow = sid * nrows + c * 2
            ebase = pl.multiple_of(crow * 128, 8)
            pltpu.sync_copy(col_h.at[pl.ds(crow, 2)], idxv)
            pltpu.sync_copy(m_h.at[pl.ds(ebase, 256)], mbuf)
            for j in range(2):
                for s in range(8):
                    v = idxv[j, pl.ds(s * 16, 16)]
                    iloc = v - lo
                    ok = (iloc >= 0) & (iloc < NHALF)
                    idxv[j, pl.ds(s * 16, 16)] = jnp.where(
                        ok, iloc, jnp.full((16,), TRASH, _i32))
                pltpu.sync_copy(mbuf.at[pl.ds(j * 128, 128)],
                                acc.at[idxv.at[j]], add=True)
            return carry

        lax.fori_loop(0, nrows // 2, chunk, 0)
        plsc.subcore_barrier()

        for q in range(10):
            r = sid * 1600 + q * 160
            pltpu.sync_copy(acc.at[pl.ds(r, 160)], wbuf)
            pltpu.sync_copy(wbuf, out_h.at[cid].at[pl.ds(r, 160)])

    return k(m, col2d)


def _dot(a, b):
    return jnp.dot(a.astype(jnp.bfloat16), b.astype(jnp.bfloat16),
                   preferred_element_type=_f32)


def _dot_exact(a, b):
    return jnp.dot(a, b, preferred_element_type=_f32,
                   precision=lax.Precision.HIGHEST)


def _ln(h, g, b):
    mu = jnp.mean(h, axis=-1, keepdims=True)
    var = jnp.mean((h - mu) ** 2, axis=-1, keepdims=True)
    return (h - mu) * lax.rsqrt(var + 1e-5) * g + b


def _mlp_refs(x, w):
    h = jnp.maximum(_dot(x, w[0][...]) + w[1][...], 0.0)
    h = jnp.maximum(_dot(h, w[2][...]) + w[3][...], 0.0)
    h = jnp.maximum(_dot(h, w[4][...]) + w[5][...], 0.0)
    h = _ln(h, w[6][...], w[7][...])
    return _dot(h, w[8][...]) + w[9][...]


def _mlp_params(p):
    return (p["l0"]["W"], p["l0"]["b"].reshape(1, -1),
            p["l1"]["W"], p["l1"]["b"].reshape(1, -1),
            p["l2"]["W"], p["l2"]["b"].reshape(1, -1),
            p["g"].reshape(1, -1), p["be"].reshape(1, -1),
            p["lo"]["W"], p["lo"]["b"].reshape(1, -1))


def _full_spec(a):
    nd = a.ndim
    return pl.BlockSpec(a.shape, lambda *i, _nd=nd: (0,) * _nd)


def _node_enc_stage(x_p, p):
    w = _mlp_params(p)

    def body(x_ref, *rest):
        o_ref = rest[-1]
        o_ref[...] = _mlp_refs(x_ref[...], rest[:-1])

    return pl.pallas_call(
        body,
        grid=(NPAD // NB,),
        in_specs=[pl.BlockSpec((NB, DIN), lambda i: (i, 0))]
        + [_full_spec(a) for a in w],
        out_specs=pl.BlockSpec((NB, H), lambda i: (i, 0)),
        out_shape=jax.ShapeDtypeStruct((NPAD, H), _f32),
    )(x_p, *w)


def _edge_enc_stage(ein, p):
    w = _mlp_params(p)
    w0 = jnp.zeros((8, H), _f32).at[:3].set(w[0])
    w = (w0,) + w[1:]

    def body(x_ref, *rest):
        o_ref = rest[-1]
        o_ref[...] = _mlp_refs(x_ref[...], rest[:-1])

    return pl.pallas_call(
        body,
        grid=(EPAD // EB,),
        in_specs=[pl.BlockSpec((EB, 8), lambda i: (i, 0))]
        + [_full_spec(a) for a in w],
        out_specs=pl.BlockSpec((EB, H), lambda i: (i, 0)),
        out_shape=jax.ShapeDtypeStruct((EPAD, H), _f32),
    )(ein, *w)


def _edge_fused_stage(hr, hc, e, be3d, u, em_p, nm1_p):
    em = _mlp_params(em_p)
    nm1 = _mlp_params(nm1_p)

    def body(hr_ref, hc_ref, e_ref, be_ref, u_ref, *rest):
        e_out, m_out = rest[-2], rest[-1]
        em_w = rest[:10]
        nm1_w = rest[10:20]
        hrv = hr_ref[...]
        hcv = hc_ref[...]
        ev = e_ref[...]
        b = be_ref[0, 0, :]
        oh = (b[:, None] == lax.broadcasted_iota(_i32, (EB, G), 1)).astype(_f32)
        ube = _dot_exact(oh, u_ref[...])
        x = jnp.concatenate([hrv, hcv, ev, ube], axis=1)
        e2 = ev + _mlp_refs(x, em_w)
        nx = jnp.concatenate([hrv, e2], axis=1)
        m = _mlp_refs(nx, nm1_w)
        e_out[...] = e2
        m_out[...] = m

    return pl.pallas_call(
        body,
        grid=(EPAD // EB,),
        in_specs=[pl.BlockSpec((EB, H), lambda i: (i, 0)),
                  pl.BlockSpec((EB, H), lambda i: (i, 0)),
                  pl.BlockSpec((EB, H), lambda i: (i, 0)),
                  pl.BlockSpec((1, 1, EB), lambda i: (i, 0, 0)),
                  _full_spec(u)]
        + [_full_spec(a) for a in em]
        + [_full_spec(a) for a in nm1],
        out_specs=[pl.BlockSpec((EB, H), lambda i: (i, 0)),
                   pl.BlockSpec((EB, H), lambda i: (i, 0))],
        out_shape=[jax.ShapeDtypeStruct((EPAD, H), _f32),
                   jax.ShapeDtypeStruct((EPAD, H), _f32)],
    )(hr, hc, e, be3d, u, *em, *nm1)


_BIG = 3.0e38


def _node_fused_stage(h, agg, batch3d, u, nm2_p):
    nm2 = _mlp_params(nm2_p)

    def body(h_ref, a_ref, b_ref, u_ref, *rest):
        (h_out, s_out, mi_out, ma_out, sq_out, cnt_out) = rest[-6:]
        w = rest[:10]
        pid = pl.program_id(0)

        @pl.when(pid == 0)
        def _init():
            s_out[...] = jnp.zeros((G, H), _f32)
            mi_out[...] = jnp.full((G, H), _BIG, _f32)
            ma_out[...] = jnp.full((G, H), -_BIG, _f32)
            sq_out[...] = jnp.zeros((G, H), _f32)
            cnt_out[...] = jnp.zeros((G, 128), _f32)

        hv = h_ref[...]
        b = b_ref[0, 0, :]
        oh = (b[:, None] == lax.broadcasted_iota(_i32, (NB, G), 1)).astype(_f32)
        ube = _dot_exact(oh, u_ref[...])
        x = jnp.concatenate([hv, a_ref[...], ube], axis=1)
        h2 = hv + _mlp_refs(x, w)
        h_out[...] = h2

        s_out[...] += lax.dot_general(oh, h2, (((0,), (0,)), ((), ())),
                                      preferred_element_type=_f32,
                                      precision=lax.Precision.HIGHEST)
        sq_out[...] += lax.dot_general(oh, h2 * h2, (((0,), (0,)), ((), ())),
                                       preferred_element_type=_f32,
                                       precision=lax.Precision.HIGHEST)
        cnt_out[...] += jnp.sum(oh, axis=0)[:, None]

        g0 = b[0]
        g1 = jnp.minimum(b[NB - 1], G - 1)

        def upd(g, carry):
            mask = b[:, None] == g
            mi = jnp.min(jnp.where(mask, h2, _BIG), axis=0)
            ma = jnp.max(jnp.where(mask, h2, -_BIG), axis=0)
            mi_out[pl.ds(g, 1), :] = jnp.minimum(mi_out[pl.ds(g, 1), :],
                                                 mi[None, :])
            ma_out[pl.ds(g, 1), :] = jnp.maximum(ma_out[pl.ds(g, 1), :],
                                                 ma[None, :])
            return carry

        lax.fori_loop(g0, g1 + 1, upd, 0)

    return pl.pallas_call(
        body,
        grid=(NPAD // NB,),
        in_specs=[pl.BlockSpec((NB, H), lambda i: (i, 0)),
                  pl.BlockSpec((NB, H), lambda i: (i, 0)),
                  pl.BlockSpec((1, 1, NB), lambda i: (i, 0, 0)),
                  _full_spec(u)]
        + [_full_spec(a) for a in nm2],
        out_specs=[pl.BlockSpec((NB, H), lambda i: (i, 0)),
                   pl.BlockSpec((G, H), lambda i: (0, 0)),
                   pl.BlockSpec((G, H), lambda i: (0, 0)),
                   pl.BlockSpec((G, H), lambda i: (0, 0)),
                   pl.BlockSpec((G, H), lambda i: (0, 0)),
                   pl.BlockSpec((G, 128), lambda i: (0, 0))],
        out_shape=[jax.ShapeDtypeStruct((NPAD, H), _f32),
                   jax.ShapeDtypeStruct((G, H), _f32),
                   jax.ShapeDtypeStruct((G, H), _f32),
                   jax.ShapeDtypeStruct((G, H), _f32),
                   jax.ShapeDtypeStruct((G, H), _f32),
                   jax.ShapeDtypeStruct((G, 128), _f32)],
    )(h, agg, batch3d, u, *nm2)


def _global_stage(u, s, mi, ma, sq, cnt, gm_p):
    gm = _mlp_params(gm_p)

    def body(u_ref, s_ref, mi_ref, ma_ref, sq_ref, cnt_ref, *rest):
        o_ref = rest[-1]
        w = rest[:10]
        uv = u_ref[...]
        sv = s_ref[...]
        c = cnt_ref[..., :H]
        nz = c > 0.0
        cc = jnp.maximum(c, 1.0)
        me = sv / cc
        mi = jnp.where(nz, mi_ref[...], 0.0)
        ma = jnp.where(nz, ma_ref[...], 0.0)
        std = sq_ref[...] / cc - me * me
        x = jnp.concatenate([uv, sv, mi, ma, std], axis=1)
        o_ref[...] = uv + _mlp_refs(x, w)

    return pl.pallas_call(
        body,
        in_specs=[_full_spec(u), _full_spec(s), _full_spec(mi),
                  _full_spec(ma), _full_spec(sq), _full_spec(cnt)]
        + [_full_spec(a) for a in gm],
        out_specs=pl.BlockSpec((G, H), lambda: (0, 0)),
        out_shape=jax.ShapeDtypeStruct((G, H), _f32),
    )(u, s, mi, ma, sq, cnt, *gm)


def _decoder_stage(pooled, ng, nb, dec_p):
    dec = _mlp_params(dec_p)

    def body(p_ref, g_ref, b_ref, *rest):
        o_ref = rest[-1]
        w = rest[:10]
        normed = _ln(p_ref[...], g_ref[...], b_ref[...])
        o_ref[...] = _mlp_refs(normed, w)

    return pl.pallas_call(
        body,
        in_specs=[_full_spec(pooled), _full_spec(ng), _full_spec(nb)]
        + [_full_spec(a) for a in dec],
        out_specs=pl.BlockSpec((G, 1), lambda: (0, 0)),
        out_shape=jax.ShapeDtypeStruct((G, 1), _f32),
    )(pooled, ng, nb, *dec)


def kernel(x, edge_index, edge_attr, batch, params):
    row = edge_index[0].astype(_i32)
    col = edge_index[1].astype(_i32)
    row_p = jnp.concatenate([row, jnp.zeros((EPAD - E,), _i32)])
    col_p = jnp.concatenate([col, jnp.full((EPAD - E,), N, _i32)])
    row2d = row_p.reshape(EROWS, 128)
    col2d = col_p.reshape(EROWS, 128)
    ea_p = jnp.concatenate([edge_attr, jnp.zeros((EPAD - E,), _f32)])
    batch_p = jnp.concatenate(
        [batch.astype(_i32), jnp.full((NPAD - N,), G, _i32)])
    x_p = jnp.concatenate([x, jnp.zeros((NPAD - N, DIN), _f32)], axis=0)

    x0 = x_p[:, 0]
    x3 = x_p[:, 3]
    bf = batch_p.astype(_f32)

    d0, d1, be = _edge_feat_stage(x0, x3, bf, row_p, col_p)
    h = _node_enc_stage(x_p, params["node_enc"])
    ein = jnp.concatenate(
        [ea_p[:, None], d0[:, None], d1[:, None], jnp.zeros((EPAD, 5), _f32)],
        axis=1)
    e = _edge_enc_stage(ein, params["edge_enc"])

    u = jnp.zeros((G, H), _f32)
    be3d = be.reshape(EPAD // EB, 1, EB)
    batch3d = batch_p.reshape(NPAD // NB, 1, NB)

    n_layers = len(params["layers"])
    s = None
    for li, lp in enumerate(params["layers"]):
        hr, hc = _gather_stage(h, row2d, col2d)
        e, m = _edge_fused_stage(hr, hc, e, be3d, u, lp["em"], lp["nm1"])
        part = _scatter_stage(m, col2d)
        agg = jnp.concatenate(
            [part[0, :NHALF], part[1, :NHALF],
             jnp.zeros((NPAD - N, H), _f32)], axis=0)
        h, s, mi, ma, sq, cnt = _node_fused_stage(h, agg, batch3d, u,
                                                  lp["nm2"])
        if li + 1 < n_layers:
            u = _global_stage(u, s, mi, ma, sq, cnt, lp["gm"])

    ng = params["norm_out"]["g"].reshape(1, H)
    nb = params["norm_out"]["b"].reshape(1, H)
    return _decoder_stage(s, ng, nb, params["decoder"])


# speedup vs baseline: 1.8441x; 1.1026x over previous
import functools

import jax
import jax.numpy as jnp
from jax import lax
from jax.experimental import pallas as pl
from jax.experimental.pallas import tpu as pltpu
from jax.experimental.pallas import tpu_sc as plsc

N = 50000
G = 256
H = 64
DIN = 128
E = 800000

NB = 2048
NPAD = 51200
EB = 2048
EPAD = 819200
EROWS = EPAD // 128
NC, NS, LANES = 2, 16, 16
NWORK = NC * NS
EW = EPAD // NWORK
CHUNK = 512
NHALF = 25000
TRASH = 25088
ACCROWS = 25600

_f32 = jnp.float32
_i32 = jnp.int32


def _sc_mesh():
    return plsc.VectorSubcoreMesh(core_axis_name="c", subcore_axis_name="s")


def _edge_feat_stage(x0, x3, bf, row_p, col_p):
    @functools.partial(
        pl.kernel,
        out_type=(jax.ShapeDtypeStruct((EPAD,), _f32),
                  jax.ShapeDtypeStruct((EPAD,), _f32),
                  jax.ShapeDtypeStruct((EPAD,), _i32)),
        mesh=_sc_mesh(),
        compiler_params=pltpu.CompilerParams(needs_layout_passes=False, use_tc_tiling_on_sc=False),
        scratch_types=[
            pltpu.VMEM((NPAD,), _f32),
            pltpu.VMEM((NPAD,), _f32),
            pltpu.VMEM((CHUNK,), _i32),
            pltpu.VMEM((CHUNK,), _i32),
            pltpu.VMEM((CHUNK,), _f32),
            pltpu.VMEM((CHUNK,), _f32),
            pltpu.VMEM((CHUNK,), _i32),
        ],
    )
    def k(x0_h, x3_h, bf_h, row_h, col_h, d0_h, d1_h, be_h,
          t0, t1, ir, ic, d0v, d1v, bev):
        wid = lax.axis_index("s") * NC + lax.axis_index("c")
        pltpu.sync_copy(x0_h, t0)
        pltpu.sync_copy(x3_h, t1)

        def chunk(c, carry):
            ebase = pl.multiple_of(wid * EW + c * CHUNK, 8)
            pltpu.sync_copy(row_h.at[pl.ds(ebase, CHUNK)], ir)
            pltpu.sync_copy(col_h.at[pl.ds(ebase, CHUNK)], ic)
            for s in range(CHUNK // 16):
                ri = ir[pl.ds(s * 16, 16)]
                ci = ic[pl.ds(s * 16, 16)]
                d0v[pl.ds(s * 16, 16)] = (plsc.load_gather(t0, [ri])
                                          - plsc.load_gather(t0, [ci]))
                d1v[pl.ds(s * 16, 16)] = (plsc.load_gather(t1, [ri])
                                          - plsc.load_gather(t1, [ci]))
            pltpu.sync_copy(d0v, d0_h.at[pl.ds(ebase, CHUNK)])
            pltpu.sync_copy(d1v, d1_h.at[pl.ds(ebase, CHUNK)])
            return carry

        lax.fori_loop(0, EW // CHUNK, chunk, 0)

        pltpu.sync_copy(bf_h, t0)

        def chunk2(c, carry):
            ebase = pl.multiple_of(wid * EW + c * CHUNK, 8)
            pltpu.sync_copy(row_h.at[pl.ds(ebase, CHUNK)], ir)
            for s in range(CHUNK // 16):
                ri = ir[pl.ds(s * 16, 16)]
                bev[pl.ds(s * 16, 16)] = plsc.load_gather(
                    t0, [ri]).astype(_i32)
            pltpu.sync_copy(bev, be_h.at[pl.ds(ebase, CHUNK)])
            return carry

        lax.fori_loop(0, EW // CHUNK, chunk2, 0)

    return k(x0, x3, bf, row_p, col_p)


GC = 256
GROWS = EW // 128
GCHUNKS = EW // GC


def _gather_stage(h, row2d, col2d):
    @functools.partial(
        pl.kernel,
        out_type=(jax.ShapeDtypeStruct((EPAD, H), _f32),
                  jax.ShapeDtypeStruct((EPAD, H), _f32)),
        mesh=_sc_mesh(),
        compiler_params=pltpu.CompilerParams(needs_layout_passes=False, use_tc_tiling_on_sc=False),
        scratch_types=[
            pltpu.VMEM((2, 128), _i32), pltpu.VMEM((2, 128), _i32),
            pltpu.VMEM((2, 128), _i32), pltpu.VMEM((2, 128), _i32),
            pltpu.VMEM((GC, H), _f32), pltpu.VMEM((GC, H), _f32),
            pltpu.VMEM((GC, H), _f32), pltpu.VMEM((GC, H), _f32),
            pltpu.SemaphoreType.DMA, pltpu.SemaphoreType.DMA,
            pltpu.SemaphoreType.DMA, pltpu.SemaphoreType.DMA,
        ],
    )
    def k(h_h, row_h, col_h, hr_h, hc_h,
          ia0, ia1, ib0, ib1, ba0, ba1, bb0, bb1,
          sg0, sg1, so0, so1):
        wid = lax.axis_index("s") * NC + lax.axis_index("c")
        ia, ib = (ia0, ia1), (ib0, ib1)
        ba, bb = (ba0, ba1), (bb0, bb1)
        sg, so = (sg0, sg1), (so0, so1)

        def crow(c):
            return wid * GROWS + c * 2

        def ebase(c):
            return pl.multiple_of(wid * EW + c * GC, 8)

        def fire_idx(c, p):
            return [pltpu.async_copy(row_h.at[pl.ds(crow(c), 2)],
                                     ia[p], sg[p]),
                    pltpu.async_copy(col_h.at[pl.ds(crow(c), 2)],
                                     ib[p], sg[p])]

        def fire_g(p):
            ds_ = []
            for j in range(2):
                ds_.append(pltpu.async_copy(
                    h_h.at[ia[p].at[j]],
                    ba[p].at[pl.ds(j * 128, 128)], sg[p]))
                ds_.append(pltpu.async_copy(
                    h_h.at[ib[p].at[j]],
                    bb[p].at[pl.ds(j * 128, 128)], sg[p]))
            return ds_

        def fire_o(c, p):
            return [pltpu.async_copy(ba[p], hr_h.at[pl.ds(ebase(c), GC)],
                                     so[p]),
                    pltpu.async_copy(bb[p], hc_h.at[pl.ds(ebase(c), GC)],
                                     so[p])]

        def body(t, carry):
            c0 = t * 2
            i0 = fire_idx(c0, 0)
            i1 = fire_idx(c0 + 1, 1)
            for d in i0:
                d.wait()
            g0 = fire_g(0)
            for d in g0:
                d.wait()
            o0 = fire_o(c0, 0)
            for d in i1:
                d.wait()
            g1 = fire_g(1)
            for d in g1:
                d.wait()
            o1 = fire_o(c0 + 1, 1)
            for d in o0 + o1:
                d.wait()
            return carry

        lax.fori_loop(0, GCHUNKS // 2, body, 0)

    return k(h, row2d, col2d)


def _scatter_stage(m, col2d):
    @functools.partial(
        pl.kernel,
        out_type=jax.ShapeDtypeStruct((2, ACCROWS, H), _f32),
        mesh=_sc_mesh(),
        compiler_params=pltpu.CompilerParams(needs_layout_passes=False, use_tc_tiling_on_sc=False),
        scratch_types=[
            pltpu.VMEM_SHARED((ACCROWS, H), _f32),
            pltpu.VMEM((2, 128), _i32),
            pltpu.VMEM((256, H), _f32),
            pltpu.VMEM((160, H), _f32),
            pltpu.SemaphoreType.DMA,
        ],
    )
    def k(m_h, col_h, out_h, acc, idxv, mbuf, wbuf, sem):
        cid = lax.axis_index("c")
        sid = lax.axis_index("s")
        lo = cid * NHALF

        def zrow(i, carry):
            for q in range(H // 16):
                wbuf[i, pl.ds(q * 16, 16)] = jnp.zeros((16,), _f32)
            return carry
        lax.fori_loop(0, 160, zrow, 0)
        for q in range(10):
            pltpu.sync_copy(wbuf, acc.at[pl.ds(sid * 1600 + q * 160, 160)])
        plsc.subcore_barrier()

        nrows = EPAD // 128 // NS

        def chunk(c, carry):
            crow = sid * nrows + c * 2
            ebase = pl.multiple_of(crow * 128, 8)
            pltpu.sync_copy(col_h.at[pl.ds(crow, 2)], idxv)
            pltpu.sync_copy(m_h.at[pl.ds(ebase, 256)], mbuf)
            for j in range(2):
                for s in range(8):
                    v = idxv[j, pl.ds(s * 16, 16)]
                    iloc = v - lo
                    ok = (iloc >= 0) & (iloc < NHALF)
                    idxv[j, pl.ds(s * 16, 16)] = jnp.where(
                        ok, iloc, jnp.full((16,), TRASH, _i32))
                pltpu.sync_copy(mbuf.at[pl.ds(j * 128, 128)],
                                acc.at[idxv.at[j]], add=True)
            return carry

        lax.fori_loop(0, nrows // 2, chunk, 0)
        plsc.subcore_barrier()

        for q in range(10):
            r = sid * 1600 + q * 160
            pltpu.sync_copy(acc.at[pl.ds(r, 160)], wbuf)
            pltpu.sync_copy(wbuf, out_h.at[cid].at[pl.ds(r, 160)])

    return k(m, col2d)


def _dot(a, b):
    return jnp.dot(a.astype(jnp.bfloat16), b.astype(jnp.bfloat16),
                   preferred_element_type=_f32)


def _dot_exact(a, b):
    return jnp.dot(a, b, preferred_element_type=_f32,
                   precision=lax.Precision.HIGHEST)


def _ln(h, g, b):
    mu = jnp.mean(h, axis=-1, keepdims=True)
    var = jnp.mean((h - mu) ** 2, axis=-1, keepdims=True)
    return (h - mu) / jnp.sqrt(var + 1e-5) * g + b


def _mlp_refs(x, w):
    h = jnp.maximum(_dot(x, w[0][...]) + w[1][...], 0.0)
    h = jnp.maximum(_dot(h, w[2][...]) + w[3][...], 0.0)
    h = jnp.maximum(_dot(h, w[4][...]) + w[5][...], 0.0)
    h = _ln(h, w[6][...], w[7][...])
    return _dot(h, w[8][...]) + w[9][...]


def _mlp_params(p):
    return (p["l0"]["W"], p["l0"]["b"].reshape(1, -1),
            p["l1"]["W"], p["l1"]["b"].reshape(1, -1),
            p["l2"]["W"], p["l2"]["b"].reshape(1, -1),
            p["g"].reshape(1, -1), p["be"].reshape(1, -1),
            p["lo"]["W"], p["lo"]["b"].reshape(1, -1))


def _full_spec(a):
    nd = a.ndim
    return pl.BlockSpec(a.shape, lambda *i, _nd=nd: (0,) * _nd)


def _node_enc_stage(x_p, p):
    w = _mlp_params(p)

    def body(x_ref, *rest):
        o_ref = rest[-1]
        o_ref[...] = _mlp_refs(x_ref[...], rest[:-1])

    return pl.pallas_call(
        body,
        grid=(NPAD // NB,),
        in_specs=[pl.BlockSpec((NB, DIN), lambda i: (i, 0))]
        + [_full_spec(a) for a in w],
        out_specs=pl.BlockSpec((NB, H), lambda i: (i, 0)),
        out_shape=jax.ShapeDtypeStruct((NPAD, H), _f32),
    )(x_p, *w)


def _edge_enc_stage(ein, p):
    w = _mlp_params(p)
    w0 = jnp.zeros((8, H), _f32).at[:3].set(w[0])
    w = (w0,) + w[1:]

    def body(x_ref, *rest):
        o_ref = rest[-1]
        o_ref[...] = _mlp_refs(x_ref[...], rest[:-1])

    return pl.pallas_call(
        body,
        grid=(EPAD // EB,),
        in_specs=[pl.BlockSpec((EB, 8), lambda i: (i, 0))]
        + [_full_spec(a) for a in w],
        out_specs=pl.BlockSpec((EB, H), lambda i: (i, 0)),
        out_shape=jax.ShapeDtypeStruct((EPAD, H), _f32),
    )(ein, *w)


def _edge_fused_stage(hr, hc, e, be3d, u, em_p, nm1_p):
    em = _mlp_params(em_p)
    nm1 = _mlp_params(nm1_p)

    def body(hr_ref, hc_ref, e_ref, be_ref, u_ref, *rest):
        e_out, m_out = rest[-2], rest[-1]
        em_w = rest[:10]
        nm1_w = rest[10:20]
        hrv = hr_ref[...]
        hcv = hc_ref[...]
        ev = e_ref[...]
        b = be_ref[0, 0, :]
        oh = (b[:, None] == lax.broadcasted_iota(_i32, (EB, G), 1)).astype(_f32)
        ube = _dot_exact(oh, u_ref[...])
        x = jnp.concatenate([hrv, hcv, ev, ube], axis=1)
        e2 = ev + _mlp_refs(x, em_w)
        nx = jnp.concatenate([hrv, e2], axis=1)
        m = _mlp_refs(nx, nm1_w)
        e_out[...] = e2
        m_out[...] = m

    return pl.pallas_call(
        body,
        grid=(EPAD // EB,),
        in_specs=[pl.BlockSpec((EB, H), lambda i: (i, 0)),
                  pl.BlockSpec((EB, H), lambda i: (i, 0)),
                  pl.BlockSpec((EB, H), lambda i: (i, 0)),
                  pl.BlockSpec((1, 1, EB), lambda i: (i, 0, 0)),
                  _full_spec(u)]
        + [_full_spec(a) for a in em]
        + [_full_spec(a) for a in nm1],
        out_specs=[pl.BlockSpec((EB, H), lambda i: (i, 0)),
                   pl.BlockSpec((EB, H), lambda i: (i, 0))],
        out_shape=[jax.ShapeDtypeStruct((EPAD, H), _f32),
                   jax.ShapeDtypeStruct((EPAD, H), _f32)],
    )(hr, hc, e, be3d, u, *em, *nm1)


_BIG = 3.0e38


def _node_fused_stage(h, agg, batch3d, u, nm2_p):
    nm2 = _mlp_params(nm2_p)

    def body(h_ref, a_ref, b_ref, u_ref, *rest):
        (h_out, s_out, mi_out, ma_out, sq_out, cnt_out) = rest[-6:]
        w = rest[:10]
        pid = pl.program_id(0)

        @pl.when(pid == 0)
        def _init():
            s_out[...] = jnp.zeros((G, H), _f32)
            mi_out[...] = jnp.full((G, H), _BIG, _f32)
            ma_out[...] = jnp.full((G, H), -_BIG, _f32)
            sq_out[...] = jnp.zeros((G, H), _f32)
            cnt_out[...] = jnp.zeros((G, 128), _f32)

        hv = h_ref[...]
        b = b_ref[0, 0, :]
        oh = (b[:, None] == lax.broadcasted_iota(_i32, (NB, G), 1)).astype(_f32)
        ube = _dot_exact(oh, u_ref[...])
        x = jnp.concatenate([hv, a_ref[...], ube], axis=1)
        h2 = hv + _mlp_refs(x, w)
        h_out[...] = h2

        s_out[...] += lax.dot_general(oh, h2, (((0,), (0,)), ((), ())),
                                      preferred_element_type=_f32,
                                      precision=lax.Precision.HIGHEST)
        sq_out[...] += lax.dot_general(oh, h2 * h2, (((0,), (0,)), ((), ())),
                                       preferred_element_type=_f32,
                                       precision=lax.Precision.HIGHEST)
        cnt_out[...] += jnp.sum(oh, axis=0)[:, None]

        g0 = b[0]
        g1 = jnp.minimum(b[NB - 1], G - 1)

        def upd(g, carry):
            mask = b[:, None] == g
            mi = jnp.min(jnp.where(mask, h2, _BIG), axis=0)
            ma = jnp.max(jnp.where(mask, h2, -_BIG), axis=0)
            mi_out[pl.ds(g, 1), :] = jnp.minimum(mi_out[pl.ds(g, 1), :],
                                                 mi[None, :])
            ma_out[pl.ds(g, 1), :] = jnp.maximum(ma_out[pl.ds(g, 1), :],
                                                 ma[None, :])
            return carry

        lax.fori_loop(g0, g1 + 1, upd, 0)

    return pl.pallas_call(
        body,
        grid=(NPAD // NB,),
        in_specs=[pl.BlockSpec((NB, H), lambda i: (i, 0)),
                  pl.BlockSpec((NB, H), lambda i: (i, 0)),
                  pl.BlockSpec((1, 1, NB), lambda i: (i, 0, 0)),
                  _full_spec(u)]
        + [_full_spec(a) for a in nm2],
        out_specs=[pl.BlockSpec((NB, H), lambda i: (i, 0)),
                   pl.BlockSpec((G, H), lambda i: (0, 0)),
                   pl.BlockSpec((G, H), lambda i: (0, 0)),
                   pl.BlockSpec((G, H), lambda i: (0, 0)),
                   pl.BlockSpec((G, H), lambda i: (0, 0)),
                   pl.BlockSpec((G, 128), lambda i: (0, 0))],
        out_shape=[jax.ShapeDtypeStruct((NPAD, H), _f32),
                   jax.ShapeDtypeStruct((G, H), _f32),
                   jax.ShapeDtypeStruct((G, H), _f32),
                   jax.ShapeDtypeStruct((G, H), _f32),
                   jax.ShapeDtypeStruct((G, H), _f32),
                   jax.ShapeDtypeStruct((G, 128), _f32)],
    )(h, agg, batch3d, u, *nm2)


def _global_stage(u, s, mi, ma, sq, cnt, gm_p):
    gm = _mlp_params(gm_p)

    def body(u_ref, s_ref, mi_ref, ma_ref, sq_ref, cnt_ref, *rest):
        o_ref = rest[-1]
        w = rest[:10]
        uv = u_ref[...]
        sv = s_ref[...]
        c = cnt_ref[..., :H]
        nz = c > 0.0
        cc = jnp.maximum(c, 1.0)
        me = sv / cc
        mi = jnp.where(nz, mi_ref[...], 0.0)
        ma = jnp.where(nz, ma_ref[...], 0.0)
        std = sq_ref[...] / cc - me * me
        x = jnp.concatenate([uv, sv, mi, ma, std], axis=1)
        o_ref[...] = uv + _mlp_refs(x, w)

    return pl.pallas_call(
        body,
        in_specs=[_full_spec(u), _full_spec(s), _full_spec(mi),
                  _full_spec(ma), _full_spec(sq), _full_spec(cnt)]
        + [_full_spec(a) for a in gm],
        out_specs=pl.BlockSpec((G, H), lambda: (0, 0)),
        out_shape=jax.ShapeDtypeStruct((G, H), _f32),
    )(u, s, mi, ma, sq, cnt, *gm)


def _decoder_stage(pooled, ng, nb, dec_p):
    dec = _mlp_params(dec_p)

    def body(p_ref, g_ref, b_ref, *rest):
        o_ref = rest[-1]
        w = rest[:10]
        normed = _ln(p_ref[...], g_ref[...], b_ref[...])
        o_ref[...] = _mlp_refs(normed, w)

    return pl.pallas_call(
        body,
        in_specs=[_full_spec(pooled), _full_spec(ng), _full_spec(nb)]
        + [_full_spec(a) for a in dec],
        out_specs=pl.BlockSpec((G, 1), lambda: (0, 0)),
        out_shape=jax.ShapeDtypeStruct((G, 1), _f32),
    )(pooled, ng, nb, *dec)


def kernel(x, edge_index, edge_attr, batch, params):
    row = edge_index[0].astype(_i32)
    col = edge_index[1].astype(_i32)
    row_p = jnp.concatenate([row, jnp.zeros((EPAD - E,), _i32)])
    col_p = jnp.concatenate([col, jnp.full((EPAD - E,), N, _i32)])
    row2d = row_p.reshape(EROWS, 128)
    col2d = col_p.reshape(EROWS, 128)
    ea_p = jnp.concatenate([edge_attr, jnp.zeros((EPAD - E,), _f32)])
    batch_p = jnp.concatenate(
        [batch.astype(_i32), jnp.full((NPAD - N,), G, _i32)])
    x_p = jnp.concatenate([x, jnp.zeros((NPAD - N, DIN), _f32)], axis=0)

    x0 = x_p[:, 0]
    x3 = x_p[:, 3]
    bf = batch_p.astype(_f32)

    d0, d1, be = _edge_feat_stage(x0, x3, bf, row_p, col_p)
    h = _node_enc_stage(x_p, params["node_enc"])
    ein = jnp.concatenate(
        [ea_p[:, None], d0[:, None], d1[:, None], jnp.zeros((EPAD, 5), _f32)],
        axis=1)
    e = _edge_enc_stage(ein, params["edge_enc"])

    u = jnp.zeros((G, H), _f32)
    be3d = be.reshape(EPAD // EB, 1, EB)
    batch3d = batch_p.reshape(NPAD // NB, 1, NB)

    n_layers = len(params["layers"])
    s = None
    for li, lp in enumerate(params["layers"]):
        hr, hc = _gather_stage(h, row2d, col2d)
        e, m = _edge_fused_stage(hr, hc, e, be3d, u, lp["em"], lp["nm1"])
        part = _scatter_stage(m, col2d)
        agg = jnp.concatenate(
            [part[0, :NHALF], part[1, :NHALF],
             jnp.zeros((NPAD - N, H), _f32)], axis=0)
        h, s, mi, ma, sq, cnt = _node_fused_stage(h, agg, batch3d, u,
                                                  lp["nm2"])
        if li + 1 < n_layers:
            u = _global_stage(u, s, mi, ma, sq, cnt, lp["gm"])

    ng = params["norm_out"]["g"].reshape(1, H)
    nb = params["norm_out"]["b"].reshape(1, H)
    return _decoder_stage(s, ng, nb, params["decoder"])
